# Initial kernel scaffold; baseline (speedup 1.0000x reference)
import jax
import jax.numpy as jnp
from jax.experimental import pallas as pl


def kernel(x, codebooks):
    raise NotImplementedError("write your pallas kernel here")


# speedup vs baseline: 5.6008x; 5.6008x over previous
import jax
import jax.numpy as jnp
from jax.experimental import pallas as pl

B, T, D = 16, 1024, 64
Q, K = 8, 1024
M = 1024
N_TOK = B * T
NB = N_TOK // M
_NEG = -3.0e38


def _split2(a):
    hi = a.astype(jnp.bfloat16)
    mid = (a - hi.astype(jnp.float32)).astype(jnp.bfloat16)
    return hi, mid


def _split3(a):
    hi = a.astype(jnp.bfloat16)
    hif = hi.astype(jnp.float32)
    mid = (a - hif).astype(jnp.bfloat16)
    lo = (a - hif - mid.astype(jnp.float32)).astype(jnp.bfloat16)
    return hi, mid, lo


def _prep_kernel(cb_ref, ch_ref, cm_ref, g_ref):
    c = cb_ref[0]
    ct = c.T
    hi = ct.astype(jnp.bfloat16)
    mid = (ct - hi.astype(jnp.float32)).astype(jnp.bfloat16)
    ch_ref[0] = hi
    cm_ref[0] = mid
    gh, gm, gl = _split3(c)
    g_ref[0] = jnp.concatenate([gh, gm, gl], axis=1)


def _prep(codebooks):
    return pl.pallas_call(
        _prep_kernel,
        grid=(Q,),
        in_specs=[pl.BlockSpec((1, K, D), lambda q: (q, 0, 0))],
        out_specs=[
            pl.BlockSpec((1, D, K), lambda q: (q, 0, 0)),
            pl.BlockSpec((1, D, K), lambda q: (q, 0, 0)),
            pl.BlockSpec((1, K, 3 * D), lambda q: (q, 0, 0)),
        ],
        out_shape=[
            jax.ShapeDtypeStruct((Q, D, K), jnp.bfloat16),
            jax.ShapeDtypeStruct((Q, D, K), jnp.bfloat16),
            jax.ShapeDtypeStruct((Q, K, 3 * D), jnp.bfloat16),
        ],
    )(codebooks)


def _search_kernel(r_ref, cb_ref, ch_ref, cm_ref, g_ref,
                   i1_ref, i2_ref, d1_ref, d2_ref, pick_ref, e1_ref):
    r = r_ref[...]
    dims = (((1,), (0,)), ((), ()))
    lane = jax.lax.broadcasted_iota(jnp.int32, (M, K), 1)
    cb = cb_ref[...]
    half = 0.5 * jnp.sum(cb * cb, axis=1)
    rh, rm = _split2(r)
    chl = ch_ref[...]
    s = (jax.lax.dot_general(rm, chl, dims, preferred_element_type=jnp.float32)
         + jax.lax.dot_general(rh, cm_ref[...], dims, preferred_element_type=jnp.float32)
         + jax.lax.dot_general(rh, chl, dims, preferred_element_type=jnp.float32))
    m = s - half[None, :]
    idx1 = jnp.argmax(m, axis=1).astype(jnp.int32)
    m2 = jnp.where(lane == idx1[:, None], jnp.float32(_NEG), m)
    idx2 = jnp.argmax(m2, axis=1).astype(jnp.int32)
    oh1 = (lane == idx1[:, None]).astype(jnp.bfloat16)
    oh2 = (lane == idx2[:, None]).astype(jnp.bfloat16)
    g = g_ref[...]
    e1c = jax.lax.dot_general(oh1, g, dims, preferred_element_type=jnp.float32)
    e2c = jax.lax.dot_general(oh2, g, dims, preferred_element_type=jnp.float32)
    e1 = (e1c[:, :D] + e1c[:, D:2 * D]) + e1c[:, 2 * D:]
    e2 = (e2c[:, :D] + e2c[:, D:2 * D]) + e2c[:, 2 * D:]
    d1 = jnp.sum((r - e1) ** 2, axis=1)
    d2 = jnp.sum((r - e2) ** 2, axis=1)
    swap = (d2 < d1) | ((d2 == d1) & (idx2 < idx1))
    idx = jnp.where(swap, idx2, idx1)
    i1_ref[0, 0, :] = idx1
    i2_ref[0, 0, :] = idx2
    d1_ref[0, 0, :] = d1
    d2_ref[0, 0, :] = d2
    pick_ref[0, 0, :] = idx
    e1_ref[...] = e1


def _apply_kernel(r_ref, qt_ref, pick_ref, g_ref,
                  ro_ref, qo_ref, loss_ref):
    i = pl.program_id(0)

    @pl.when(i == 0)
    def _init():
        loss_ref[...] = jnp.zeros((1, 1), jnp.float32)

    r = r_ref[...]
    dims = (((1,), (0,)), ((), ()))
    lane = jax.lax.broadcasted_iota(jnp.int32, (M, K), 1)
    idx = pick_ref[0, 0, :]
    oh = (lane == idx[:, None]).astype(jnp.bfloat16)
    ec = jax.lax.dot_general(oh, g_ref[...], dims,
                             preferred_element_type=jnp.float32)
    e = (ec[:, :D] + ec[:, D:2 * D]) + ec[:, 2 * D:]
    rn = r - e
    ro_ref[...] = rn
    qo_ref[...] = qt_ref[...] + e
    loss_ref[...] += jnp.sum((rn - e) ** 2).reshape(1, 1)


def _search(r, cb, ch, cm, gcat):
    return pl.pallas_call(
        _search_kernel,
        grid=(NB,),
        in_specs=[
            pl.BlockSpec((M, D), lambda i: (i, 0)),
            pl.BlockSpec((K, D), lambda i: (0, 0)),
            pl.BlockSpec((D, K), lambda i: (0, 0)),
            pl.BlockSpec((D, K), lambda i: (0, 0)),
            pl.BlockSpec((K, 3 * D), lambda i: (0, 0)),
        ],
        out_specs=[
            pl.BlockSpec((1, 1, M), lambda i: (i, 0, 0)),
            pl.BlockSpec((1, 1, M), lambda i: (i, 0, 0)),
            pl.BlockSpec((1, 1, M), lambda i: (i, 0, 0)),
            pl.BlockSpec((1, 1, M), lambda i: (i, 0, 0)),
            pl.BlockSpec((1, 1, M), lambda i: (i, 0, 0)),
            pl.BlockSpec((M, D), lambda i: (i, 0)),
        ],
        out_shape=[
            jax.ShapeDtypeStruct((NB, 1, M), jnp.int32),
            jax.ShapeDtypeStruct((NB, 1, M), jnp.int32),
            jax.ShapeDtypeStruct((NB, 1, M), jnp.float32),
            jax.ShapeDtypeStruct((NB, 1, M), jnp.float32),
            jax.ShapeDtypeStruct((NB, 1, M), jnp.int32),
            jax.ShapeDtypeStruct((N_TOK, D), jnp.float32),
        ],
    )(r, cb, ch, cm, gcat)


def _apply(r, quant, pick, gcat):
    return pl.pallas_call(
        _apply_kernel,
        grid=(NB,),
        in_specs=[
            pl.BlockSpec((M, D), lambda i: (i, 0)),
            pl.BlockSpec((M, D), lambda i: (i, 0)),
            pl.BlockSpec((1, 1, M), lambda i: (i, 0, 0)),
            pl.BlockSpec((K, 3 * D), lambda i: (0, 0)),
        ],
        out_specs=[
            pl.BlockSpec((M, D), lambda i: (i, 0)),
            pl.BlockSpec((M, D), lambda i: (i, 0)),
            pl.BlockSpec((1, 1), lambda i: (0, 0)),
        ],
        out_shape=[
            jax.ShapeDtypeStruct((N_TOK, D), jnp.float32),
            jax.ShapeDtypeStruct((N_TOK, D), jnp.float32),
            jax.ShapeDtypeStruct((1, 1), jnp.float32),
        ],
    )(r, quant, pick, gcat)


def kernel(x, codebooks):
    xf = x.reshape(N_TOK, D)
    ch, cm, gcat = _prep(codebooks)
    r = xf
    quant = jnp.zeros_like(xf)
    codes_list = []
    loss = jnp.float32(0.0)
    for q in range(Q):
        _i1, _i2, _d1, _d2, pick, _e1 = _search(
            r, codebooks[q], ch[q], cm[q], gcat[q])
        r, quant, lpart = _apply(r, quant, pick, gcat[q])
        codes_list.append(pick.reshape(B, T))
        loss = loss + lpart[0, 0]
    quantized = (xf + (quant - xf)).reshape(B, T, D)
    lossv = loss * jnp.float32(0.25 / (B * T * D))
    return (quantized, lossv, *codes_list)
